# Initial kernel scaffold; baseline (speedup 1.0000x reference)
import functools

import jax
import jax.numpy as jnp
from jax import lax
from jax.experimental import pallas as pl
from jax.experimental.pallas import tpu as pltpu
from jax.experimental.pallas import tpu_sc as plsc

_RC = 3.0
_N_NODES = 100000
_N_EDGES = 6400000
_E0_HALF = 2.0 * ((1.0 / _RC) ** 12 - (1.0 / _RC) ** 6)

_NC = 2
_NS = 16
_L = 16
_NW = _NC * _NS
_EPT = _N_EDGES // _NW
_CHUNK = 8000
_NCHUNK = _EPT // _CHUNK
_PAD = 100352

_mesh = plsc.VectorSubcoreMesh(core_axis_name="c", subcore_axis_name="s")


@functools.partial(
    pl.kernel,
    mesh=_mesh,
    out_type=jax.ShapeDtypeStruct((_NW, _PAD), jnp.float32),
    scratch_types=[
        pltpu.VMEM((_CHUNK,), jnp.float32),
        pltpu.VMEM((_CHUNK,), jnp.int32),
        pltpu.VMEM((_PAD,), jnp.float32),
    ],
)
def _sc_lj_scatter(dist_hbm, idx_hbm, out_hbm, dist_v, idx_v, acc_v):
    wid = lax.axis_index("s") * _NC + lax.axis_index("c")
    base = wid * _EPT

    zero = jnp.zeros((_L,), jnp.float32)

    @plsc.parallel_loop(0, _PAD // (_L * 8))
    def _zero(i):
        for u in range(8):
            acc_v[pl.ds(i * (_L * 8) + u * _L, _L)] = zero

    for c in range(_NCHUNK):
        off = base + c * _CHUNK
        pltpu.sync_copy(dist_hbm.at[pl.ds(off, _CHUNK)], dist_v)
        pltpu.sync_copy(idx_hbm.at[pl.ds(off, _CHUNK)], idx_v)

        def _body(j, carry):
            d = dist_v[pl.ds(j * _L, _L)]
            ix = idx_v[pl.ds(j * _L, _L)]
            r = 1.0 / d
            r2 = r * r
            r6 = r2 * r2 * r2
            en = (r6 * r6 - r6) * 2.0 - _E0_HALF
            plsc.addupdate_scatter(acc_v, [ix], en)
            return carry

        lax.fori_loop(0, _CHUNK // _L, _body, 0)

    pltpu.sync_copy(acc_v, out_hbm.at[wid])


def _tc_reduce_body(p_ref, o_ref):
    o_ref[...] = jnp.sum(p_ref[...], axis=0)


_BLK = 1024
_tc_reduce = pl.pallas_call(
    _tc_reduce_body,
    grid=(_PAD // _BLK,),
    in_specs=[pl.BlockSpec((_NW, _BLK), lambda i: (0, i))],
    out_specs=pl.BlockSpec((_BLK,), lambda i: (i,)),
    out_shape=jax.ShapeDtypeStruct((_PAD,), jnp.float32),
)


def kernel(dist, ind_1, ind_2):
    del ind_1
    idx = ind_2[:, 0].astype(jnp.int32)
    partials = _sc_lj_scatter(dist, idx)
    en = _tc_reduce(partials)
    return en[:_N_NODES]


# speedup vs baseline: 19.7291x; 19.7291x over previous
import functools

import jax
import jax.numpy as jnp
from jax import lax
from jax.experimental import pallas as pl
from jax.experimental.pallas import tpu as pltpu
from jax.experimental.pallas import tpu_sc as plsc

_RC = 3.0
_N_NODES = 100000
_N_EDGES = 6400000
_E0_HALF = 2.0 * ((1.0 / _RC) ** 12 - (1.0 / _RC) ** 6)

_NC = 2
_NS = 16
_L = 16
_NW = _NC * _NS
_EPT = _N_EDGES // _NW
_CHUNK = 8000
_NCHUNK = _EPT // _CHUNK
_PAD = 100352

_mesh = plsc.VectorSubcoreMesh(core_axis_name="c", subcore_axis_name="s")


@functools.partial(
    pl.kernel,
    mesh=_mesh,
    out_type=jax.ShapeDtypeStruct((_NW, _PAD), jnp.float32),
    scratch_types=[
        pltpu.VMEM((_CHUNK,), jnp.float32),
        pltpu.VMEM((_CHUNK,), jnp.int32),
        pltpu.VMEM((_PAD,), jnp.float32),
    ],
    compiler_params=pltpu.CompilerParams(needs_layout_passes=False),
)
def _sc_lj_scatter(dist_hbm, idx_hbm, out_hbm, dist_v, idx_v, acc_v):
    wid = lax.axis_index("s") * _NC + lax.axis_index("c")
    base = wid * _EPT

    zero = jnp.zeros((_L,), jnp.float32)

    @plsc.parallel_loop(0, _PAD // (_L * 8))
    def _zero(i):
        for u in range(8):
            acc_v[pl.ds(i * (_L * 8) + u * _L, _L)] = zero

    for c in range(_NCHUNK):
        off = base + c * _CHUNK
        pltpu.sync_copy(dist_hbm.at[pl.ds(off, _CHUNK)], dist_v)
        pltpu.sync_copy(idx_hbm.at[pl.ds(off, _CHUNK)], idx_v)

        def _body(j, carry):
            d = dist_v[pl.ds(j * _L, _L)]
            ix = idx_v[pl.ds(j * _L, _L)]
            r = 1.0 / d
            r2 = r * r
            r6 = r2 * r2 * r2
            en = (r6 * r6 - r6) * 2.0 - _E0_HALF
            plsc.addupdate_scatter(acc_v, [ix], en)
            return carry

        lax.fori_loop(0, _CHUNK // _L, _body, 0)

    pltpu.sync_copy(acc_v, out_hbm.at[wid])


def _tc_reduce_body(p_ref, o_ref):
    o_ref[...] = jnp.sum(p_ref[...], axis=0)


_BLK = 1024
_tc_reduce = pl.pallas_call(
    _tc_reduce_body,
    grid=(_PAD // _BLK,),
    in_specs=[pl.BlockSpec((_NW, _BLK), lambda i: (0, i))],
    out_specs=pl.BlockSpec((_BLK,), lambda i: (i,)),
    out_shape=jax.ShapeDtypeStruct((_PAD,), jnp.float32),
)


def kernel(dist, ind_1, ind_2):
    del ind_1
    idx = ind_2[:, 0].astype(jnp.int32)
    partials = _sc_lj_scatter(dist, idx)
    en = _tc_reduce(partials)
    return en[:_N_NODES]


# speedup vs baseline: 39.9031x; 2.0225x over previous
import functools

import jax
import jax.numpy as jnp
from jax import lax
from jax.experimental import pallas as pl
from jax.experimental.pallas import tpu as pltpu
from jax.experimental.pallas import tpu_sc as plsc

_RC = 3.0
_N_NODES = 100000
_N_EDGES = 6400000
_E0_HALF = 2.0 * ((1.0 / _RC) ** 12 - (1.0 / _RC) ** 6)

_NC = 2
_NS = 16
_L = 16
_NW = _NC * _NS
_EPT = _N_EDGES // _NW
_CHUNK = 8000
_NCHUNK = _EPT // _CHUNK
_PAD = 100352

_mesh = plsc.VectorSubcoreMesh(core_axis_name="c", subcore_axis_name="s")


@functools.partial(
    pl.kernel,
    mesh=_mesh,
    out_type=jax.ShapeDtypeStruct((_NW, _PAD), jnp.float32),
    scratch_types=[
        pltpu.VMEM((_CHUNK,), jnp.float32),
        pltpu.VMEM((_CHUNK,), jnp.int32),
        pltpu.VMEM((_PAD,), jnp.float32),
    ],
    compiler_params=pltpu.CompilerParams(needs_layout_passes=False),
)
def _sc_lj_scatter(dist_hbm, idx_hbm, out_hbm, dist_v, idx_v, acc_v):
    wid = lax.axis_index("s") * _NC + lax.axis_index("c")
    base = wid * _EPT

    zero = jnp.zeros((_L,), jnp.float32)

    @plsc.parallel_loop(0, _PAD // (_L * 8))
    def _zero(i):
        for u in range(8):
            acc_v[pl.ds(i * (_L * 8) + u * _L, _L)] = zero

    for c in range(_NCHUNK):
        off = base + c * _CHUNK
        pltpu.sync_copy(dist_hbm.at[pl.ds(off, _CHUNK)], dist_v)
        pltpu.sync_copy(idx_hbm.at[pl.ds(off, _CHUNK)], idx_v)

        @plsc.parallel_loop(0, _CHUNK // _L, unroll=8)
        def _body(j):
            d = dist_v[pl.ds(j * _L, _L)]
            ix = idx_v[pl.ds(j * _L, _L)]
            r = 1.0 / d
            r2 = r * r
            r6 = r2 * r2 * r2
            en = (r6 * r6 - r6) * 2.0 - _E0_HALF
            plsc.addupdate_scatter(acc_v, [ix], en)

    pltpu.sync_copy(acc_v, out_hbm.at[wid])


def _tc_reduce_body(p_ref, o_ref):
    o_ref[...] = jnp.sum(p_ref[...], axis=0)


_BLK = 1024
_tc_reduce = pl.pallas_call(
    _tc_reduce_body,
    grid=(_PAD // _BLK,),
    in_specs=[pl.BlockSpec((_NW, _BLK), lambda i: (0, i))],
    out_specs=pl.BlockSpec((_BLK,), lambda i: (i,)),
    out_shape=jax.ShapeDtypeStruct((_PAD,), jnp.float32),
)


def kernel(dist, ind_1, ind_2):
    del ind_1
    idx = ind_2[:, 0].astype(jnp.int32)
    partials = _sc_lj_scatter(dist, idx)
    en = _tc_reduce(partials)
    return en[:_N_NODES]
